# baseline (speedup 1.0000x reference)
import jax
import jax.numpy as jnp
from jax.experimental import pallas as pl
from jax.experimental.pallas import tpu as pltpu

_GAMMA = 0.1
_BAND = 150
_BIG = 1e10
_KL_FREE_BITS = 0.5
_W_KL, _W_RECON, _W_AUX, _W_TRANS = 1.0, 1.0, 0.1, 0.5


def _softmin3(a, b, c, gamma):
    m = jnp.minimum(jnp.minimum(a, b), c)
    s = (jnp.exp((m - a) * (1.0 / gamma))
         + jnp.exp((m - b) * (1.0 / gamma))
         + jnp.exp((m - c) * (1.0 / gamma)))
    return m - gamma * jnp.log(s)


def _dp_kernel(x_ref, y_ref, o_ref, d1_ref, d2_ref, ys_ref):
    N = x_ref.shape[2]
    R = 48
    x = x_ref[0]
    ii = jax.lax.broadcasted_iota(jnp.int32, (R, N), 1)
    ii128 = jax.lax.broadcasted_iota(jnp.int32, (192, 128), 1)

    x0 = x_ref[0, :, 0:1]
    y0 = y_ref[0, :, 0:1]
    ad0 = jnp.abs(x0 - y0)
    dp0 = ad0[0:R] + ad0[R:2 * R] + ad0[2 * R:3 * R] + ad0[3 * R:4 * R]
    d1_ref[...] = jnp.where(ii == 0, dp0, _BIG)
    d2_ref[...] = jnp.full((R, N), _BIG, jnp.float32)
    ys_ref[...] = jnp.concatenate(
        [y0, jnp.zeros((192, N - 1), jnp.float32)], axis=1)

    def step(p, _):
        s = jnp.minimum(p // 128, (N // 128) - 1)
        base = pl.multiple_of(s * 128, 128)
        slab = y_ref[0, :, pl.ds(base, 128)]
        newcol = jnp.sum(jnp.where(ii128 + base == p, slab, 0.0), axis=1,
                         keepdims=True)
        ys = jnp.concatenate([newcol, ys_ref[:, :N - 1]], axis=1)
        ys_ref[...] = ys
        ad = jnp.abs(x - ys)
        Dp = ad[0:R] + ad[R:2 * R] + ad[2 * R:3 * R] + ad[3 * R:4 * R]
        d1 = d1_ref[...]
        d2 = d2_ref[...]
        big_col = jnp.full((R, 1), _BIG, jnp.float32)
        sh_d1 = jnp.concatenate([big_col, d1[:, :N - 1]], axis=1)
        sh_d2 = jnp.concatenate([big_col, d2[:, :N - 1]], axis=1)
        r = Dp + _softmin3(sh_d2, sh_d1, d1, _GAMMA)
        jj = p - ii
        valid = (jj >= 0) & (jj < N) & (jnp.abs(ii - jj) <= _BAND)
        d_cur = jnp.where(valid, r, _BIG)
        d2_ref[...] = d1
        d1_ref[...] = d_cur
        return 0

    jax.lax.fori_loop(1, 2 * N - 1, step, 0)
    o_ref[0] = d1_ref[:, N - 128:]


def _loss_kernel(v_ref, mu_ref, lv_ref, ptc_ref, gtt_ref, att_ref, o_ref):
    v = v_ref[...]
    vnorm = v[0:32] - 0.5 * (v[32:64] + v[64:96])
    recon = jnp.sum(vnorm) * (1.0 / (32.0 * 32.0))

    mu = mu_ref[...]
    lv = lv_ref[...]
    kl_div = -0.5 * jnp.sum(1.0 + lv - mu * mu - jnp.exp(lv), axis=1)
    kl = jnp.mean(jnp.maximum(kl_div - _KL_FREE_BITS, 0.0))

    gtt = gtt_ref[...]
    gt_trans = jnp.sum(jnp.abs(gtt[:, 1:] - gtt[:, :-1]), axis=1)
    ptc = ptc_ref[...][:, 0]
    aux = jnp.mean((ptc - gt_trans) ** 2)

    att = jax.nn.sigmoid((att_ref[...] - 0.5) * 10.0)
    pred_soft = jnp.sum(jnp.abs(att[:, 1:] - att[:, :-1]), axis=1)
    trans = jnp.mean((pred_soft - gt_trans) ** 2)

    total = _W_RECON * recon + _W_KL * kl + _W_AUX * aux + _W_TRANS * trans
    lane = jax.lax.broadcasted_iota(jnp.int32, (1, 8), 1)
    out = (jnp.where(lane == 0, total, 0.0)
           + jnp.where(lane == 1, recon, 0.0)
           + jnp.where(lane == 2, kl, 0.0)
           + jnp.where(lane == 3, aux, 0.0)
           + jnp.where(lane == 4, trans, 0.0))
    o_ref[...] = out


def kernel(action_trajectory, style_mu, style_logvar,
           predicted_transition_count, ground_truth, interpret=False):
    at = action_trajectory
    gt = ground_truth
    B, N, D = at.shape
    xs = jnp.concatenate([at, at, gt], axis=0)
    ys = jnp.concatenate([gt, at, gt], axis=0)
    xc = xs.transpose(2, 0, 1).reshape(D, 2, 48, N).transpose(1, 0, 2, 3)
    xc = xc.reshape(2, 4 * 48, N)
    yc = ys.transpose(2, 0, 1).reshape(D, 2, 48, N).transpose(1, 0, 2, 3)
    yc = yc.reshape(2, 4 * 48, N)

    last = pl.pallas_call(
        _dp_kernel,
        grid=(2,),
        in_specs=[
            pl.BlockSpec((1, 192, N), lambda h: (h, 0, 0)),
            pl.BlockSpec((1, 192, N), lambda h: (h, 0, 0)),
        ],
        out_specs=pl.BlockSpec((1, 48, 128), lambda h: (h, 0, 0)),
        out_shape=jax.ShapeDtypeStruct((2, 48, 128), jnp.float32),
        scratch_shapes=[
            pltpu.VMEM((48, N), jnp.float32),
            pltpu.VMEM((48, N), jnp.float32),
            pltpu.VMEM((192, N), jnp.float32),
        ],
        compiler_params=pltpu.CompilerParams(
            dimension_semantics=("arbitrary",),
        ),
        name="sdtw_dp",
        interpret=interpret,
    )(xc, yc)
    v = last[:, :, -1].reshape(96, 1)

    out = pl.pallas_call(
        _loss_kernel,
        out_shape=jax.ShapeDtypeStruct((1, 8), jnp.float32),
        name="cvae_losses",
        interpret=interpret,
    )(v, style_mu, style_logvar, predicted_transition_count,
      ground_truth[..., 2], action_trajectory[..., 2])
    return out[0, :5]


# speedup vs baseline: 1.6740x; 1.6740x over previous
import jax
import jax.numpy as jnp
from jax.experimental import pallas as pl
from jax.experimental.pallas import tpu as pltpu

_GAMMA = 0.1
_BAND = 150
_BIG = 1e10
_KL_FREE_BITS = 0.5
_W_KL, _W_RECON, _W_AUX, _W_TRANS = 1.0, 1.0, 0.1, 0.5

_N = 512
_V = 256
_R = 48
_F = 4


def _softmin3(a, b, c):
    m = jnp.minimum(jnp.minimum(a, b), c)
    s = (jnp.exp((m - a) * (1.0 / _GAMMA))
         + jnp.exp((m - b) * (1.0 / _GAMMA))
         + jnp.exp((m - c) * (1.0 / _GAMMA)))
    return m - _GAMMA * jnp.log(s)


def _sum4(ad):
    return (ad[0:_R] + ad[_R:2 * _R] + ad[2 * _R:3 * _R] + ad[3 * _R:4 * _R])


def _dp_kernel(x_ref, y_ref, o_ref, d1_ref, d2_ref, ys_ref, xs_ref):
    ii_u = jax.lax.broadcasted_iota(jnp.int32, (_R, _V), 1)
    ii128 = jax.lax.broadcasted_iota(jnp.int32, (192, 128), 1)
    big_col = jnp.full((_R, 1), _BIG, jnp.float32)

    def extract_col(ref, idx):
        s = jnp.minimum(idx // 128, (_N // 128) - 1)
        base = pl.multiple_of(s * 128, 128)
        slab = ref[0, :, pl.ds(base, 128)]
        return jnp.sum(jnp.where(ii128 + base == idx, slab, 0.0), axis=1,
                       keepdims=True)

    def shr(a, fill_col):
        return jnp.concatenate([fill_col, a[:, :_V - 1]], axis=1)

    x0 = x_ref[0, :, 0:1]
    y0 = y_ref[0, :, 0:1]
    dp0 = _sum4(jnp.abs(x0 - y0))
    d1_ref[...] = jnp.where(ii_u == 0, dp0, _BIG)
    d2_ref[...] = jnp.full((_R, _V), _BIG, jnp.float32)
    ys_ref[...] = jnp.concatenate(
        [y0, jnp.zeros((192, _V - 1), jnp.float32)], axis=1)
    xs_ref[...] = x_ref[0, :, 0:_V]

    def hold_step(p, base):
        d1 = d1_ref[...]
        d2 = d2_ref[...]
        ycol = extract_col(y_ref, p - base)
        ys = jnp.concatenate([ycol, ys_ref[:, :_V - 1]], axis=1)
        ys_ref[...] = ys
        dp = _sum4(jnp.abs(xs_ref[...] - ys))
        r = dp + _softmin3(shr(d2, big_col), shr(d1, big_col), d1)
        i_ = ii_u + base
        valid = ((jnp.abs(2 * i_ - p) <= _BAND) & (i_ <= p)
                 & (i_ >= p - (_N - 1)))
        d_cur = jnp.where(valid, r, _BIG)
        d2_ref[...] = d1
        d1_ref[...] = d_cur

    def phase1_body(p, _):
        hold_step(p, 0)
        return 0

    def pair_body(k, _):
        p = 151 + 2 * k
        bs = k + 1
        d1 = d1_ref[...]
        d2 = d2_ref[...]
        ys = ys_ref[...]
        xcol = extract_col(x_ref, bs + (_V - 1))
        xs = jnp.concatenate([xs_ref[:, 1:], xcol], axis=1)
        xs_ref[...] = xs
        dp_a = _sum4(jnp.abs(xs - ys))
        shl_d1 = jnp.concatenate([d1[:, 1:], big_col], axis=1)
        r_a = dp_a + _softmin3(d2, d1, shl_d1)
        i_ = ii_u + bs
        valid_a = (jnp.abs(2 * i_ - p) <= _BAND) & (i_ <= p)
        da = jnp.where(valid_a, r_a, _BIG)
        ycol = extract_col(y_ref, k + 151)
        ys2 = jnp.concatenate([ycol, ys[:, :_V - 1]], axis=1)
        ys_ref[...] = ys2
        dp_b = _sum4(jnp.abs(xs - ys2))
        r_b = dp_b + _softmin3(d1, shr(da, big_col), da)
        valid_b = (jnp.abs(2 * i_ - (p + 1)) <= _BAND) & (i_ <= p + 1)
        db = jnp.where(valid_b, r_b, _BIG)
        d2_ref[...] = da
        d1_ref[...] = db
        return 0

    def phase3_body(p, _):
        hold_step(p, _N - _V)
        return 0

    jax.lax.fori_loop(1, 151, phase1_body, 0)
    jax.lax.fori_loop(0, 256, pair_body, 0)
    jax.lax.fori_loop(663, 2 * _N - 1, phase3_body, 0)
    o_ref[0] = d1_ref[:, _V - 128:]


def _loss_kernel(v_ref, mu_ref, lv_ref, ptc_ref, gtt_ref, att_ref, o_ref):
    v = v_ref[...]
    vnorm = v[0:32] - 0.5 * (v[32:64] + v[64:96])
    recon = jnp.sum(vnorm) * (1.0 / (32.0 * 32.0))

    mu = mu_ref[...]
    lv = lv_ref[...]
    kl_div = -0.5 * jnp.sum(1.0 + lv - mu * mu - jnp.exp(lv), axis=1)
    kl = jnp.mean(jnp.maximum(kl_div - _KL_FREE_BITS, 0.0))

    gtt = gtt_ref[...]
    gt_trans = jnp.sum(jnp.abs(gtt[:, 1:] - gtt[:, :-1]), axis=1)
    ptc = ptc_ref[...][:, 0]
    aux = jnp.mean((ptc - gt_trans) ** 2)

    att = jax.nn.sigmoid((att_ref[...] - 0.5) * 10.0)
    pred_soft = jnp.sum(jnp.abs(att[:, 1:] - att[:, :-1]), axis=1)
    trans = jnp.mean((pred_soft - gt_trans) ** 2)

    total = _W_RECON * recon + _W_KL * kl + _W_AUX * aux + _W_TRANS * trans
    lane = jax.lax.broadcasted_iota(jnp.int32, (1, 8), 1)
    out = (jnp.where(lane == 0, total, 0.0)
           + jnp.where(lane == 1, recon, 0.0)
           + jnp.where(lane == 2, kl, 0.0)
           + jnp.where(lane == 3, aux, 0.0)
           + jnp.where(lane == 4, trans, 0.0))
    o_ref[...] = out


def kernel(action_trajectory, style_mu, style_logvar,
           predicted_transition_count, ground_truth, interpret=False):
    at = action_trajectory
    gt = ground_truth
    B, N, D = at.shape
    xs = jnp.concatenate([at, at, gt], axis=0)
    ys = jnp.concatenate([gt, at, gt], axis=0)
    xc = xs.transpose(2, 0, 1).reshape(D, 2, _R, N).transpose(1, 0, 2, 3)
    xc = xc.reshape(2, _F * _R, N)
    yc = ys.transpose(2, 0, 1).reshape(D, 2, _R, N).transpose(1, 0, 2, 3)
    yc = yc.reshape(2, _F * _R, N)

    last = pl.pallas_call(
        _dp_kernel,
        grid=(2,),
        in_specs=[
            pl.BlockSpec((1, 192, N), lambda h: (h, 0, 0)),
            pl.BlockSpec((1, 192, N), lambda h: (h, 0, 0)),
        ],
        out_specs=pl.BlockSpec((1, _R, 128), lambda h: (h, 0, 0)),
        out_shape=jax.ShapeDtypeStruct((2, _R, 128), jnp.float32),
        scratch_shapes=[
            pltpu.VMEM((_R, _V), jnp.float32),
            pltpu.VMEM((_R, _V), jnp.float32),
            pltpu.VMEM((192, _V), jnp.float32),
            pltpu.VMEM((192, _V), jnp.float32),
        ],
        compiler_params=pltpu.CompilerParams(
            dimension_semantics=("arbitrary",),
        ),
        name="sdtw_dp",
        interpret=interpret,
    )(xc, yc)
    v = last[:, :, -1].reshape(96, 1)

    out = pl.pallas_call(
        _loss_kernel,
        out_shape=jax.ShapeDtypeStruct((1, 8), jnp.float32),
        name="cvae_losses",
        interpret=interpret,
    )(v, style_mu, style_logvar, predicted_transition_count,
      ground_truth[..., 2], action_trajectory[..., 2])
    return out[0, :5]


# speedup vs baseline: 2.0240x; 1.2091x over previous
import jax
import jax.numpy as jnp
from jax.experimental import pallas as pl
from jax.experimental.pallas import tpu as pltpu

_GAMMA = 0.1
_BAND = 150
_BIG = 1e10
_KL_FREE_BITS = 0.5
_W_KL, _W_RECON, _W_AUX, _W_TRANS = 1.0, 1.0, 0.1, 0.5

_N = 512
_V = 256
_R = 96
_F = 4


def _softmin3(a, b, c):
    m = jnp.minimum(jnp.minimum(a, b), c)
    s = (jnp.exp((m - a) * (1.0 / _GAMMA))
         + jnp.exp((m - b) * (1.0 / _GAMMA))
         + jnp.exp((m - c) * (1.0 / _GAMMA)))
    return m - _GAMMA * jnp.log(s)


def _sum4(ad):
    return (ad[0:_R] + ad[_R:2 * _R] + ad[2 * _R:3 * _R] + ad[3 * _R:4 * _R])


def _dp_kernel(x_ref, y_ref, o_ref, d1_ref, d2_ref, ys_ref, xs_ref):
    ii_u = jax.lax.broadcasted_iota(jnp.int32, (_R, _V), 1)
    ii128 = jax.lax.broadcasted_iota(jnp.int32, (4 * _R, 128), 1)
    big_col = jnp.full((_R, 1), _BIG, jnp.float32)

    def extract_col(ref, idx):
        s = jnp.minimum(idx // 128, (_N // 128) - 1)
        base = pl.multiple_of(s * 128, 128)
        slab = ref[:, pl.ds(base, 128)]
        return jnp.sum(jnp.where(ii128 + base == idx, slab, 0.0), axis=1,
                       keepdims=True)

    def shr(a, fill_col):
        return jnp.concatenate([fill_col, a[:, :_V - 1]], axis=1)

    x0 = x_ref[:, 0:1]
    y0 = y_ref[:, 0:1]
    dp0 = _sum4(jnp.abs(x0 - y0))
    d1_ref[...] = jnp.where(ii_u == 0, dp0, _BIG)
    d2_ref[...] = jnp.full((_R, _V), _BIG, jnp.float32)
    ys_ref[...] = jnp.concatenate(
        [y0, jnp.zeros((4 * _R, _V - 1), jnp.float32)], axis=1)
    xs_ref[...] = x_ref[:, 0:_V]

    def hold_step(p, base):
        d1 = d1_ref[...]
        d2 = d2_ref[...]
        ycol = extract_col(y_ref, p - base)
        ys = jnp.concatenate([ycol, ys_ref[:, :_V - 1]], axis=1)
        ys_ref[...] = ys
        dp = _sum4(jnp.abs(xs_ref[...] - ys))
        r = dp + _softmin3(shr(d2, big_col), shr(d1, big_col), d1)
        i_ = ii_u + base
        valid = ((jnp.abs(2 * i_ - p) <= _BAND) & (i_ <= p)
                 & (i_ >= p - (_N - 1)))
        d_cur = jnp.where(valid, r, _BIG)
        d2_ref[...] = d1
        d1_ref[...] = d_cur

    def phase1_body(p, _):
        hold_step(p, 0)
        return 0

    def pair_body(k, _):
        p = 151 + 2 * k
        bs = k + 1
        d1 = d1_ref[...]
        d2 = d2_ref[...]
        ys = ys_ref[...]
        xcol = extract_col(x_ref, bs + (_V - 1))
        xs = jnp.concatenate([xs_ref[:, 1:], xcol], axis=1)
        xs_ref[...] = xs
        dp_a = _sum4(jnp.abs(xs - ys))
        shl_d1 = jnp.concatenate([d1[:, 1:], big_col], axis=1)
        r_a = dp_a + _softmin3(d2, d1, shl_d1)
        i_ = ii_u + bs
        valid_a = (jnp.abs(2 * i_ - p) <= _BAND) & (i_ <= p)
        da = jnp.where(valid_a, r_a, _BIG)
        ycol = extract_col(y_ref, k + 151)
        ys2 = jnp.concatenate([ycol, ys[:, :_V - 1]], axis=1)
        ys_ref[...] = ys2
        dp_b = _sum4(jnp.abs(xs - ys2))
        r_b = dp_b + _softmin3(d1, shr(da, big_col), da)
        valid_b = (jnp.abs(2 * i_ - (p + 1)) <= _BAND) & (i_ <= p + 1)
        db = jnp.where(valid_b, r_b, _BIG)
        d2_ref[...] = da
        d1_ref[...] = db
        return 0

    def phase3_body(p, _):
        hold_step(p, _N - _V)
        return 0

    jax.lax.fori_loop(1, 151, phase1_body, 0)
    jax.lax.fori_loop(0, 256, pair_body, 0)
    jax.lax.fori_loop(663, 2 * _N - 1, phase3_body, 0)
    o_ref[...] = d1_ref[:, _V - 128:]


def _loss_kernel(v_ref, mu_ref, lv_ref, ptc_ref, gtt_ref, att_ref, o_ref):
    v = v_ref[...]
    vnorm = v[0:32] - 0.5 * (v[32:64] + v[64:96])
    recon = jnp.sum(vnorm) * (1.0 / (32.0 * 32.0))

    mu = mu_ref[...]
    lv = lv_ref[...]
    kl_div = -0.5 * jnp.sum(1.0 + lv - mu * mu - jnp.exp(lv), axis=1)
    kl = jnp.mean(jnp.maximum(kl_div - _KL_FREE_BITS, 0.0))

    gtt = gtt_ref[...]
    gt_trans = jnp.sum(jnp.abs(gtt[:, 1:] - gtt[:, :-1]), axis=1)
    ptc = ptc_ref[...][:, 0]
    aux = jnp.mean((ptc - gt_trans) ** 2)

    att = jax.nn.sigmoid((att_ref[...] - 0.5) * 10.0)
    pred_soft = jnp.sum(jnp.abs(att[:, 1:] - att[:, :-1]), axis=1)
    trans = jnp.mean((pred_soft - gt_trans) ** 2)

    total = _W_RECON * recon + _W_KL * kl + _W_AUX * aux + _W_TRANS * trans
    lane = jax.lax.broadcasted_iota(jnp.int32, (1, 8), 1)
    out = (jnp.where(lane == 0, total, 0.0)
           + jnp.where(lane == 1, recon, 0.0)
           + jnp.where(lane == 2, kl, 0.0)
           + jnp.where(lane == 3, aux, 0.0)
           + jnp.where(lane == 4, trans, 0.0))
    o_ref[...] = out


def kernel(action_trajectory, style_mu, style_logvar,
           predicted_transition_count, ground_truth, interpret=False):
    at = action_trajectory
    gt = ground_truth
    B, N, D = at.shape
    xs = jnp.concatenate([at, at, gt], axis=0)
    ys = jnp.concatenate([gt, at, gt], axis=0)
    xc = xs.transpose(2, 0, 1).reshape(_F * _R, N)
    yc = ys.transpose(2, 0, 1).reshape(_F * _R, N)

    last = pl.pallas_call(
        _dp_kernel,
        out_shape=jax.ShapeDtypeStruct((_R, 128), jnp.float32),
        scratch_shapes=[
            pltpu.VMEM((_R, _V), jnp.float32),
            pltpu.VMEM((_R, _V), jnp.float32),
            pltpu.VMEM((4 * _R, _V), jnp.float32),
            pltpu.VMEM((4 * _R, _V), jnp.float32),
        ],
        name="sdtw_dp",
        interpret=interpret,
    )(xc, yc)
    v = last[:, -1].reshape(96, 1)

    out = pl.pallas_call(
        _loss_kernel,
        out_shape=jax.ShapeDtypeStruct((1, 8), jnp.float32),
        name="cvae_losses",
        interpret=interpret,
    )(v, style_mu, style_logvar, predicted_transition_count,
      ground_truth[..., 2], action_trajectory[..., 2])
    return out[0, :5]


# speedup vs baseline: 2.0715x; 1.0235x over previous
import jax
import jax.numpy as jnp
from jax.experimental import pallas as pl
from jax.experimental.pallas import tpu as pltpu

_GAMMA = 0.1
_BAND = 150
_BIG = 1e10
_KL_FREE_BITS = 0.5
_W_KL, _W_RECON, _W_AUX, _W_TRANS = 1.0, 1.0, 0.1, 0.5

_N = 512
_V = 256
_R = 96
_F = 4


def _softmin3(a, b, c):
    m = jnp.minimum(jnp.minimum(a, b), c)
    s = jnp.exp(m - a) + jnp.exp(m - b) + jnp.exp(m - c)
    return m - jnp.log(s)


def _sum4(ad):
    return (ad[0:_R] + ad[_R:2 * _R] + ad[2 * _R:3 * _R] + ad[3 * _R:4 * _R])


def _dp_kernel(x_ref, y_ref, o_ref, d1_ref, d2_ref, ys_ref, xs_ref):
    ii_u = jax.lax.broadcasted_iota(jnp.int32, (_R, _V), 1)
    ii128 = jax.lax.broadcasted_iota(jnp.int32, (4 * _R, 128), 1)
    big_col = jnp.full((_R, 1), _BIG, jnp.float32)

    def extract_col(ref, idx):
        s = jnp.minimum(idx // 128, (_N // 128) - 1)
        base = pl.multiple_of(s * 128, 128)
        slab = ref[:, pl.ds(base, 128)]
        return jnp.sum(jnp.where(ii128 + base == idx, slab, 0.0), axis=1,
                       keepdims=True)

    def shr(a, fill_col):
        return jnp.concatenate([fill_col, a[:, :_V - 1]], axis=1)

    x0 = x_ref[:, 0:1]
    y0 = y_ref[:, 0:1]
    dp0 = _sum4(jnp.abs(x0 - y0)) * (1.0 / _GAMMA)
    d1_ref[...] = jnp.where(ii_u == 0, dp0, _BIG)
    d2_ref[...] = jnp.full((_R, _V), _BIG, jnp.float32)
    ys_ref[...] = jnp.concatenate(
        [y0, jnp.zeros((4 * _R, _V - 1), jnp.float32)], axis=1)
    xs_ref[...] = x_ref[:, 0:_V]

    def hold_step(p, base):
        d1 = d1_ref[...]
        d2 = d2_ref[...]
        ycol = extract_col(y_ref, p - base)
        ys = jnp.concatenate([ycol, ys_ref[:, :_V - 1]], axis=1)
        ys_ref[...] = ys
        dp = _sum4(jnp.abs(xs_ref[...] - ys)) * (1.0 / _GAMMA)
        r = dp + _softmin3(shr(d2, big_col), shr(d1, big_col), d1)
        lo = jnp.maximum(jnp.maximum((p - 149) // 2, p - (_N - 1)), 0) - base
        hi = jnp.minimum(jnp.minimum((p + 150) // 2, p), _N - 1) - base
        valid = (ii_u >= lo) & (ii_u <= hi)
        d_cur = jnp.where(valid, r, _BIG)
        d2_ref[...] = d1
        d1_ref[...] = d_cur

    def phase1_body(p, _):
        hold_step(p, 0)
        return 0

    def pair_body(k, _):
        p = 151 + 2 * k
        bs = k + 1
        d1 = d1_ref[...]
        d2 = d2_ref[...]
        ys = ys_ref[...]
        xcol = extract_col(x_ref, bs + (_V - 1))
        xs = jnp.concatenate([xs_ref[:, 1:], xcol], axis=1)
        xs_ref[...] = xs
        dp_a = _sum4(jnp.abs(xs - ys)) * (1.0 / _GAMMA)
        shl_d1 = jnp.concatenate([d1[:, 1:], big_col], axis=1)
        r_a = dp_a + _softmin3(d2, d1, shl_d1)
        lo_a = jnp.maximum((p - 149) // 2, 0) - bs
        hi_a = jnp.minimum((p + 150) // 2, p) - bs
        valid_a = (ii_u >= lo_a) & (ii_u <= hi_a)
        da = jnp.where(valid_a, r_a, _BIG)
        ycol = extract_col(y_ref, k + 151)
        ys2 = jnp.concatenate([ycol, ys[:, :_V - 1]], axis=1)
        ys_ref[...] = ys2
        dp_b = _sum4(jnp.abs(xs - ys2)) * (1.0 / _GAMMA)
        r_b = dp_b + _softmin3(d1, shr(da, big_col), da)
        lo_b = jnp.maximum((p - 148) // 2, 0) - bs
        hi_b = jnp.minimum((p + 151) // 2, p + 1) - bs
        valid_b = (ii_u >= lo_b) & (ii_u <= hi_b)
        db = jnp.where(valid_b, r_b, _BIG)
        d2_ref[...] = da
        d1_ref[...] = db
        return 0

    def phase3_body(p, _):
        hold_step(p, _N - _V)
        return 0

    jax.lax.fori_loop(1, 151, phase1_body, 0)
    jax.lax.fori_loop(0, 256, pair_body, 0)
    jax.lax.fori_loop(663, 2 * _N - 1, phase3_body, 0)
    o_ref[...] = d1_ref[:, _V - 128:]


def _loss_kernel(v_ref, mu_ref, lv_ref, ptc_ref, gtt_ref, att_ref, o_ref):
    v = v_ref[...] * _GAMMA
    vnorm = v[0:32] - 0.5 * (v[32:64] + v[64:96])
    recon = jnp.sum(vnorm) * (1.0 / (32.0 * 32.0))

    mu = mu_ref[...]
    lv = lv_ref[...]
    kl_div = -0.5 * jnp.sum(1.0 + lv - mu * mu - jnp.exp(lv), axis=1)
    kl = jnp.mean(jnp.maximum(kl_div - _KL_FREE_BITS, 0.0))

    gtt = gtt_ref[...]
    gt_trans = jnp.sum(jnp.abs(gtt[:, 1:] - gtt[:, :-1]), axis=1)
    ptc = ptc_ref[...][:, 0]
    aux = jnp.mean((ptc - gt_trans) ** 2)

    att = jax.nn.sigmoid((att_ref[...] - 0.5) * 10.0)
    pred_soft = jnp.sum(jnp.abs(att[:, 1:] - att[:, :-1]), axis=1)
    trans = jnp.mean((pred_soft - gt_trans) ** 2)

    total = _W_RECON * recon + _W_KL * kl + _W_AUX * aux + _W_TRANS * trans
    lane = jax.lax.broadcasted_iota(jnp.int32, (1, 8), 1)
    out = (jnp.where(lane == 0, total, 0.0)
           + jnp.where(lane == 1, recon, 0.0)
           + jnp.where(lane == 2, kl, 0.0)
           + jnp.where(lane == 3, aux, 0.0)
           + jnp.where(lane == 4, trans, 0.0))
    o_ref[...] = out


def kernel(action_trajectory, style_mu, style_logvar,
           predicted_transition_count, ground_truth, interpret=False):
    at = action_trajectory
    gt = ground_truth
    B, N, D = at.shape
    xs = jnp.concatenate([at, at, gt], axis=0)
    ys = jnp.concatenate([gt, at, gt], axis=0)
    xc = xs.transpose(2, 0, 1).reshape(_F * _R, N)
    yc = ys.transpose(2, 0, 1).reshape(_F * _R, N)

    last = pl.pallas_call(
        _dp_kernel,
        out_shape=jax.ShapeDtypeStruct((_R, 128), jnp.float32),
        scratch_shapes=[
            pltpu.VMEM((_R, _V), jnp.float32),
            pltpu.VMEM((_R, _V), jnp.float32),
            pltpu.VMEM((4 * _R, _V), jnp.float32),
            pltpu.VMEM((4 * _R, _V), jnp.float32),
        ],
        name="sdtw_dp",
        interpret=interpret,
    )(xc, yc)
    v = last[:, -1].reshape(96, 1)

    out = pl.pallas_call(
        _loss_kernel,
        out_shape=jax.ShapeDtypeStruct((1, 8), jnp.float32),
        name="cvae_losses",
        interpret=interpret,
    )(v, style_mu, style_logvar, predicted_transition_count,
      ground_truth[..., 2], action_trajectory[..., 2])
    return out[0, :5]


# speedup vs baseline: 2.2038x; 1.0639x over previous
import jax
import jax.numpy as jnp
from jax.experimental import pallas as pl
from jax.experimental.pallas import tpu as pltpu

_GAMMA = 0.1
_BAND = 150
_BIG = 1e10
_KL_FREE_BITS = 0.5
_W_KL, _W_RECON, _W_AUX, _W_TRANS = 1.0, 1.0, 0.1, 0.5

_N = 512
_V = 256
_R = 96
_F = 4


def _softmin3(a, b, c):
    m = jnp.minimum(jnp.minimum(a, b), c)
    s = jnp.exp(m - a) + jnp.exp(m - b) + jnp.exp(m - c)
    return m - jnp.log(s)


def _sum4(ad):
    return (ad[0:_R] + ad[_R:2 * _R] + ad[2 * _R:3 * _R] + ad[3 * _R:4 * _R])


def _dp_kernel(x_ref, y_ref, o_ref, d1_ref, d2_ref, ys_ref, xs_ref):
    ii_u = jax.lax.broadcasted_iota(jnp.int32, (_R, _V), 1)
    ii128 = jax.lax.broadcasted_iota(jnp.int32, (4 * _R, 128), 1)
    big_col = jnp.full((_R, 1), _BIG, jnp.float32)

    def extract_col(ref, idx):
        s = jnp.minimum(idx // 128, (_N // 128) - 1)
        base = pl.multiple_of(s * 128, 128)
        slab = ref[:, pl.ds(base, 128)]
        return jnp.sum(jnp.where(ii128 + base == idx, slab, 0.0), axis=1,
                       keepdims=True)

    def shr(a, fill_col):
        return jnp.concatenate([fill_col, a[:, :_V - 1]], axis=1)

    x0 = x_ref[:, 0:1]
    y0 = y_ref[:, 0:1]
    dp0 = _sum4(jnp.abs(x0 - y0)) * (1.0 / _GAMMA)
    d1_ref[...] = jnp.where(ii_u == 0, dp0, _BIG)
    d2_ref[...] = jnp.full((_R, _V), _BIG, jnp.float32)
    ys_ref[...] = jnp.concatenate(
        [y0, jnp.zeros((4 * _R, _V - 1), jnp.float32)], axis=1)
    xs_ref[...] = x_ref[:, 0:_V]

    def band_mask(p, base):
        lo = jnp.maximum(jnp.maximum((p - 149) // 2, p - (_N - 1)), 0) - base
        hi = jnp.minimum(jnp.minimum((p + 150) // 2, p), _N - 1) - base
        return (ii_u >= lo) & (ii_u <= hi)

    def hold2_step(p, base):
        d1 = d1_ref[...]
        d2 = d2_ref[...]
        ys0 = ys_ref[...]
        xs = xs_ref[...]
        yca = extract_col(y_ref, p - base)
        ycb = extract_col(y_ref, p + 1 - base)
        ysa = jnp.concatenate([yca, ys0[:, :_V - 1]], axis=1)
        ysb = jnp.concatenate([ycb, yca, ys0[:, :_V - 2]], axis=1)
        ys_ref[...] = ysb
        dpa = _sum4(jnp.abs(xs - ysa)) * (1.0 / _GAMMA)
        dpb = _sum4(jnp.abs(xs - ysb)) * (1.0 / _GAMMA)
        sh_d1 = shr(d1, big_col)
        ra = dpa + _softmin3(shr(d2, big_col), sh_d1, d1)
        da = jnp.where(band_mask(p, base), ra, _BIG)
        rb = dpb + _softmin3(sh_d1, shr(da, big_col), da)
        db = jnp.where(band_mask(p + 1, base), rb, _BIG)
        d2_ref[...] = da
        d1_ref[...] = db

    def phase1_body(t, _):
        hold2_step(1 + 2 * t, 0)
        return 0

    def pair2_body(k2, _):
        k = 2 * k2
        p1 = 151 + 2 * k
        bs1 = k + 1
        d1 = d1_ref[...]
        d2 = d2_ref[...]
        ys0 = ys_ref[...]
        xs0 = xs_ref[...]
        xcol1 = extract_col(x_ref, bs1 + (_V - 1))
        xcol2 = extract_col(x_ref, bs1 + _V)
        ycol1 = extract_col(y_ref, k + 151)
        ycol2 = extract_col(y_ref, k + 152)
        xs1 = jnp.concatenate([xs0[:, 1:], xcol1], axis=1)
        xs2 = jnp.concatenate([xs0[:, 2:], xcol1, xcol2], axis=1)
        xs_ref[...] = xs2
        ys1 = jnp.concatenate([ycol1, ys0[:, :_V - 1]], axis=1)
        ys2 = jnp.concatenate([ycol2, ycol1, ys0[:, :_V - 2]], axis=1)
        ys_ref[...] = ys2
        dp_a1 = _sum4(jnp.abs(xs1 - ys0)) * (1.0 / _GAMMA)
        r_a1 = dp_a1 + _softmin3(
            d2, d1, jnp.concatenate([d1[:, 1:], big_col], axis=1))
        da1 = jnp.where(band_mask(p1, bs1), r_a1, _BIG)
        dp_b1 = _sum4(jnp.abs(xs1 - ys1)) * (1.0 / _GAMMA)
        r_b1 = dp_b1 + _softmin3(d1, shr(da1, big_col), da1)
        db1 = jnp.where(band_mask(p1 + 1, bs1), r_b1, _BIG)
        dp_a2 = _sum4(jnp.abs(xs2 - ys1)) * (1.0 / _GAMMA)
        r_a2 = dp_a2 + _softmin3(
            da1, db1, jnp.concatenate([db1[:, 1:], big_col], axis=1))
        da2 = jnp.where(band_mask(p1 + 2, bs1 + 1), r_a2, _BIG)
        dp_b2 = _sum4(jnp.abs(xs2 - ys2)) * (1.0 / _GAMMA)
        r_b2 = dp_b2 + _softmin3(db1, shr(da2, big_col), da2)
        db2 = jnp.where(band_mask(p1 + 3, bs1 + 1), r_b2, _BIG)
        d2_ref[...] = da2
        d1_ref[...] = db2
        return 0

    def phase3_body(t, _):
        hold2_step(663 + 2 * t, _N - _V)
        return 0

    jax.lax.fori_loop(0, 75, phase1_body, 0)
    jax.lax.fori_loop(0, 128, pair2_body, 0)
    jax.lax.fori_loop(0, 180, phase3_body, 0)
    o_ref[...] = d1_ref[:, _V - 128:]


def _loss_kernel(v_ref, mu_ref, lv_ref, ptc_ref, gtt_ref, att_ref, o_ref):
    v = v_ref[...] * _GAMMA
    vnorm = v[0:32] - 0.5 * (v[32:64] + v[64:96])
    recon = jnp.sum(vnorm) * (1.0 / (32.0 * 32.0))

    mu = mu_ref[...]
    lv = lv_ref[...]
    kl_div = -0.5 * jnp.sum(1.0 + lv - mu * mu - jnp.exp(lv), axis=1)
    kl = jnp.mean(jnp.maximum(kl_div - _KL_FREE_BITS, 0.0))

    gtt = gtt_ref[...]
    gt_trans = jnp.sum(jnp.abs(gtt[:, 1:] - gtt[:, :-1]), axis=1)
    ptc = ptc_ref[...][:, 0]
    aux = jnp.mean((ptc - gt_trans) ** 2)

    att = jax.nn.sigmoid((att_ref[...] - 0.5) * 10.0)
    pred_soft = jnp.sum(jnp.abs(att[:, 1:] - att[:, :-1]), axis=1)
    trans = jnp.mean((pred_soft - gt_trans) ** 2)

    total = _W_RECON * recon + _W_KL * kl + _W_AUX * aux + _W_TRANS * trans
    lane = jax.lax.broadcasted_iota(jnp.int32, (1, 8), 1)
    out = (jnp.where(lane == 0, total, 0.0)
           + jnp.where(lane == 1, recon, 0.0)
           + jnp.where(lane == 2, kl, 0.0)
           + jnp.where(lane == 3, aux, 0.0)
           + jnp.where(lane == 4, trans, 0.0))
    o_ref[...] = out


def kernel(action_trajectory, style_mu, style_logvar,
           predicted_transition_count, ground_truth, interpret=False):
    at = action_trajectory
    gt = ground_truth
    B, N, D = at.shape
    xs = jnp.concatenate([at, at, gt], axis=0)
    ys = jnp.concatenate([gt, at, gt], axis=0)
    xc = xs.transpose(2, 0, 1).reshape(_F * _R, N)
    yc = ys.transpose(2, 0, 1).reshape(_F * _R, N)

    last = pl.pallas_call(
        _dp_kernel,
        out_shape=jax.ShapeDtypeStruct((_R, 128), jnp.float32),
        scratch_shapes=[
            pltpu.VMEM((_R, _V), jnp.float32),
            pltpu.VMEM((_R, _V), jnp.float32),
            pltpu.VMEM((4 * _R, _V), jnp.float32),
            pltpu.VMEM((4 * _R, _V), jnp.float32),
        ],
        name="sdtw_dp",
        interpret=interpret,
    )(xc, yc)
    v = last[:, -1].reshape(96, 1)

    out = pl.pallas_call(
        _loss_kernel,
        out_shape=jax.ShapeDtypeStruct((1, 8), jnp.float32),
        name="cvae_losses",
        interpret=interpret,
    )(v, style_mu, style_logvar, predicted_transition_count,
      ground_truth[..., 2], action_trajectory[..., 2])
    return out[0, :5]


# speedup vs baseline: 2.6316x; 1.1941x over previous
import jax
import jax.numpy as jnp
from jax.experimental import pallas as pl
from jax.experimental.pallas import tpu as pltpu

_GAMMA = 0.1
_BAND = 150
_BIG = 1e10
_KL_FREE_BITS = 0.5
_W_KL, _W_RECON, _W_AUX, _W_TRANS = 1.0, 1.0, 0.1, 0.5

_N = 512
_V = 256
_R = 96
_F = 4


def _softmin3(a, b, c):
    m = jnp.minimum(jnp.minimum(a, b), c)
    s = jnp.exp(m - a) + jnp.exp(m - b) + jnp.exp(m - c)
    return m - jnp.log(s)


def _sum4(ad):
    return (ad[0:_R] + ad[_R:2 * _R] + ad[2 * _R:3 * _R] + ad[3 * _R:4 * _R])


def _dp_kernel(x_ref, y_ref, o_ref, d1_ref, d2_ref, ys_ref, xs_ref):
    ii_u = jax.lax.broadcasted_iota(jnp.int32, (_R, _V), 1)
    ii128 = jax.lax.broadcasted_iota(jnp.int32, (4 * _R, 128), 1)
    big_col = jnp.full((_R, 1), _BIG, jnp.float32)

    def extract_col(ref, idx):
        s = jnp.minimum(idx // 128, (_N // 128) - 1)
        base = pl.multiple_of(s * 128, 128)
        slab = ref[:, pl.ds(base, 128)]
        return jnp.sum(jnp.where(ii128 + base == idx, slab,
                                 jnp.bfloat16(0.0)), axis=1, keepdims=True)

    def shr(a, fill_col):
        return jnp.concatenate([fill_col, a[:, :_V - 1]], axis=1)

    x0 = x_ref[:, 0:1]
    y0 = y_ref[:, 0:1]
    dp0 = _sum4(jnp.abs(x0 - y0)).astype(jnp.float32) * (1.0 / _GAMMA)
    d1_ref[...] = jnp.where(ii_u == 0, dp0, _BIG)
    d2_ref[...] = jnp.full((_R, _V), _BIG, jnp.float32)
    ys_ref[...] = jnp.concatenate(
        [y0, jnp.zeros((4 * _R, _V - 1), jnp.bfloat16)], axis=1)
    xs_ref[...] = x_ref[:, 0:_V]

    def band_mask(p, base):
        lo = jnp.maximum(jnp.maximum((p - 149) // 2, p - (_N - 1)), 0) - base
        hi = jnp.minimum(jnp.minimum((p + 150) // 2, p), _N - 1) - base
        return (ii_u >= lo) & (ii_u <= hi)

    def hold2_step(p, base):
        d1 = d1_ref[...]
        d2 = d2_ref[...]
        ys0 = ys_ref[...]
        xs = xs_ref[...]
        yca = extract_col(y_ref, p - base)
        ycb = extract_col(y_ref, p + 1 - base)
        ysa = jnp.concatenate([yca, ys0[:, :_V - 1]], axis=1)
        ysb = jnp.concatenate([ycb, yca, ys0[:, :_V - 2]], axis=1)
        ys_ref[...] = ysb
        dpa = _sum4(jnp.abs(xs - ysa)).astype(jnp.float32) * (1.0 / _GAMMA)
        dpb = _sum4(jnp.abs(xs - ysb)).astype(jnp.float32) * (1.0 / _GAMMA)
        sh_d1 = shr(d1, big_col)
        ra = dpa + _softmin3(shr(d2, big_col), sh_d1, d1)
        da = jnp.where(band_mask(p, base), ra, _BIG)
        rb = dpb + _softmin3(sh_d1, shr(da, big_col), da)
        db = jnp.where(band_mask(p + 1, base), rb, _BIG)
        d2_ref[...] = da
        d1_ref[...] = db

    def phase1_body(t, _):
        hold2_step(1 + 2 * t, 0)
        return 0

    def pair2_body(k2, _):
        k = 2 * k2
        p1 = 151 + 2 * k
        bs1 = k + 1
        d1 = d1_ref[...]
        d2 = d2_ref[...]
        ys0 = ys_ref[...]
        xs0 = xs_ref[...]
        xcol1 = extract_col(x_ref, bs1 + (_V - 1))
        xcol2 = extract_col(x_ref, bs1 + _V)
        ycol1 = extract_col(y_ref, k + 151)
        ycol2 = extract_col(y_ref, k + 152)
        xs1 = jnp.concatenate([xs0[:, 1:], xcol1], axis=1)
        xs2 = jnp.concatenate([xs0[:, 2:], xcol1, xcol2], axis=1)
        xs_ref[...] = xs2
        ys1 = jnp.concatenate([ycol1, ys0[:, :_V - 1]], axis=1)
        ys2 = jnp.concatenate([ycol2, ycol1, ys0[:, :_V - 2]], axis=1)
        ys_ref[...] = ys2
        dp_a1 = _sum4(jnp.abs(xs1 - ys0)).astype(jnp.float32) * (1.0 / _GAMMA)
        r_a1 = dp_a1 + _softmin3(
            d2, d1, jnp.concatenate([d1[:, 1:], big_col], axis=1))
        da1 = jnp.where(band_mask(p1, bs1), r_a1, _BIG)
        dp_b1 = _sum4(jnp.abs(xs1 - ys1)).astype(jnp.float32) * (1.0 / _GAMMA)
        r_b1 = dp_b1 + _softmin3(d1, shr(da1, big_col), da1)
        db1 = jnp.where(band_mask(p1 + 1, bs1), r_b1, _BIG)
        dp_a2 = _sum4(jnp.abs(xs2 - ys1)).astype(jnp.float32) * (1.0 / _GAMMA)
        r_a2 = dp_a2 + _softmin3(
            da1, db1, jnp.concatenate([db1[:, 1:], big_col], axis=1))
        da2 = jnp.where(band_mask(p1 + 2, bs1 + 1), r_a2, _BIG)
        dp_b2 = _sum4(jnp.abs(xs2 - ys2)).astype(jnp.float32) * (1.0 / _GAMMA)
        r_b2 = dp_b2 + _softmin3(db1, shr(da2, big_col), da2)
        db2 = jnp.where(band_mask(p1 + 3, bs1 + 1), r_b2, _BIG)
        d2_ref[...] = da2
        d1_ref[...] = db2
        return 0

    def phase3_body(t, _):
        hold2_step(663 + 2 * t, _N - _V)
        return 0

    jax.lax.fori_loop(0, 75, phase1_body, 0)
    jax.lax.fori_loop(0, 128, pair2_body, 0)
    jax.lax.fori_loop(0, 180, phase3_body, 0)
    o_ref[...] = d1_ref[:, _V - 128:]


def _loss_kernel(v_ref, mu_ref, lv_ref, ptc_ref, gtt_ref, att_ref, o_ref):
    v = v_ref[...] * _GAMMA
    vnorm = v[0:32] - 0.5 * (v[32:64] + v[64:96])
    recon = jnp.sum(vnorm) * (1.0 / (32.0 * 32.0))

    mu = mu_ref[...]
    lv = lv_ref[...]
    kl_div = -0.5 * jnp.sum(1.0 + lv - mu * mu - jnp.exp(lv), axis=1)
    kl = jnp.mean(jnp.maximum(kl_div - _KL_FREE_BITS, 0.0))

    gtt = gtt_ref[...]
    gt_trans = jnp.sum(jnp.abs(gtt[:, 1:] - gtt[:, :-1]), axis=1)
    ptc = ptc_ref[...][:, 0]
    aux = jnp.mean((ptc - gt_trans) ** 2)

    att = jax.nn.sigmoid((att_ref[...] - 0.5) * 10.0)
    pred_soft = jnp.sum(jnp.abs(att[:, 1:] - att[:, :-1]), axis=1)
    trans = jnp.mean((pred_soft - gt_trans) ** 2)

    total = _W_RECON * recon + _W_KL * kl + _W_AUX * aux + _W_TRANS * trans
    lane = jax.lax.broadcasted_iota(jnp.int32, (1, 8), 1)
    out = (jnp.where(lane == 0, total, 0.0)
           + jnp.where(lane == 1, recon, 0.0)
           + jnp.where(lane == 2, kl, 0.0)
           + jnp.where(lane == 3, aux, 0.0)
           + jnp.where(lane == 4, trans, 0.0))
    o_ref[...] = out


def kernel(action_trajectory, style_mu, style_logvar,
           predicted_transition_count, ground_truth, interpret=False):
    at = action_trajectory
    gt = ground_truth
    B, N, D = at.shape
    xs = jnp.concatenate([at, at, gt], axis=0)
    ys = jnp.concatenate([gt, at, gt], axis=0)
    xc = xs.transpose(2, 0, 1).reshape(_F * _R, N).astype(jnp.bfloat16)
    yc = ys.transpose(2, 0, 1).reshape(_F * _R, N).astype(jnp.bfloat16)

    last = pl.pallas_call(
        _dp_kernel,
        out_shape=jax.ShapeDtypeStruct((_R, 128), jnp.float32),
        scratch_shapes=[
            pltpu.VMEM((_R, _V), jnp.float32),
            pltpu.VMEM((_R, _V), jnp.float32),
            pltpu.VMEM((4 * _R, _V), jnp.bfloat16),
            pltpu.VMEM((4 * _R, _V), jnp.bfloat16),
        ],
        name="sdtw_dp",
        interpret=interpret,
    )(xc, yc)
    v = last[:, -1].reshape(96, 1)

    out = pl.pallas_call(
        _loss_kernel,
        out_shape=jax.ShapeDtypeStruct((1, 8), jnp.float32),
        name="cvae_losses",
        interpret=interpret,
    )(v, style_mu, style_logvar, predicted_transition_count,
      ground_truth[..., 2], action_trajectory[..., 2])
    return out[0, :5]


# speedup vs baseline: 2.7313x; 1.0379x over previous
import jax
import jax.numpy as jnp
from jax.experimental import pallas as pl
from jax.experimental.pallas import tpu as pltpu

_GAMMA = 0.1
_BAND = 150
_BIG = 1e10
_KL_FREE_BITS = 0.5
_W_KL, _W_RECON, _W_AUX, _W_TRANS = 1.0, 1.0, 0.1, 0.5

_N = 512
_V = 256
_R = 96
_F = 4


def _softmin3(a, b, c):
    m = jnp.minimum(jnp.minimum(a, b), c)
    s = jnp.exp(m - a) + jnp.exp(m - b) + jnp.exp(m - c)
    return m - jnp.log(s)


def _sum4(ad):
    return (ad[0:_R] + ad[_R:2 * _R] + ad[2 * _R:3 * _R] + ad[3 * _R:4 * _R])


def _dp_kernel(x_ref, y_ref, o_ref, d1_ref, d2_ref, ys_ref, xs_ref):
    ii_u = jax.lax.broadcasted_iota(jnp.int32, (_R, _V), 1)
    ii128 = jax.lax.broadcasted_iota(jnp.int32, (4 * _R, 128), 1)
    big_col = jnp.full((_R, 1), _BIG, jnp.float32)

    def extract_col(ref, idx):
        s = jnp.minimum(idx // 128, (_N // 128) - 1)
        base = pl.multiple_of(s * 128, 128)
        slab = ref[:, pl.ds(base, 128)]
        return jnp.sum(jnp.where(ii128 + base == idx, slab,
                                 jnp.bfloat16(0.0)), axis=1, keepdims=True)

    def shr(a, fill_col):
        return jnp.concatenate([fill_col, a[:, :_V - 1]], axis=1)

    x0 = x_ref[:, 0:1]
    y0 = y_ref[:, 0:1]
    dp0 = _sum4(jnp.abs(x0 - y0)).astype(jnp.float32) * (1.0 / _GAMMA)
    d1_ref[...] = jnp.where(ii_u == 0, dp0, _BIG)
    d2_ref[...] = jnp.full((_R, _V), _BIG, jnp.float32)
    ys_ref[...] = jnp.concatenate(
        [y0, jnp.zeros((4 * _R, _V - 1), jnp.bfloat16)], axis=1)
    xs_ref[...] = x_ref[:, 0:_V]

    def band_mask(p, base):
        lo = jnp.maximum(jnp.maximum((p - 149) // 2, p - (_N - 1)), 0) - base
        hi = jnp.minimum(jnp.minimum((p + 150) // 2, p), _N - 1) - base
        return (ii_u >= lo) & (ii_u <= hi)

    def hold2_step(p, base, s0, s1):
        w = s1 - s0
        d1 = d1_ref[:, s0:s1]
        d2 = d2_ref[:, s0:s1]
        ys0 = ys_ref[...]
        xs = xs_ref[:, s0:s1]
        yca = extract_col(y_ref, p - base)
        ycb = extract_col(y_ref, p + 1 - base)
        ysa = jnp.concatenate([yca, ys0[:, :_V - 1]], axis=1)
        ysb = jnp.concatenate([ycb, yca, ys0[:, :_V - 2]], axis=1)
        ys_ref[...] = ysb
        dpa = (_sum4(jnp.abs(xs - ysa[:, s0:s1])).astype(jnp.float32)
               * (1.0 / _GAMMA))
        dpb = (_sum4(jnp.abs(xs - ysb[:, s0:s1])).astype(jnp.float32)
               * (1.0 / _GAMMA))
        bc = big_col
        shn = lambda a: jnp.concatenate([bc, a[:, :w - 1]], axis=1)
        ii_n = jax.lax.broadcasted_iota(jnp.int32, (_R, w), 1) + s0
        def mask(q):
            lo = jnp.maximum(jnp.maximum((q - 149) // 2, q - (_N - 1)),
                             0) - base
            hi = jnp.minimum(jnp.minimum((q + 150) // 2, q), _N - 1) - base
            return (ii_n >= lo) & (ii_n <= hi)
        sh_d1 = shn(d1)
        ra = dpa + _softmin3(shn(d2), sh_d1, d1)
        da = jnp.where(mask(p), ra, _BIG)
        rb = dpb + _softmin3(sh_d1, shn(da), da)
        db = jnp.where(mask(p + 1), rb, _BIG)
        d2_ref[:, s0:s1] = da
        d1_ref[:, s0:s1] = db

    def phase1n_body(t, _):
        hold2_step(1 + 2 * t, 0, 0, 128)
        return 0

    def phase1_body(t, _):
        hold2_step(1 + 2 * t, 0, 0, _V)
        return 0

    def pair2_body(k2, _):
        k = 2 * k2
        p1 = 151 + 2 * k
        bs1 = k + 1
        d1 = d1_ref[...]
        d2 = d2_ref[...]
        ys0 = ys_ref[...]
        xs0 = xs_ref[...]
        xcol1 = extract_col(x_ref, bs1 + (_V - 1))
        xcol2 = extract_col(x_ref, bs1 + _V)
        ycol1 = extract_col(y_ref, k + 151)
        ycol2 = extract_col(y_ref, k + 152)
        xs1 = jnp.concatenate([xs0[:, 1:], xcol1], axis=1)
        xs2 = jnp.concatenate([xs0[:, 2:], xcol1, xcol2], axis=1)
        xs_ref[...] = xs2
        ys1 = jnp.concatenate([ycol1, ys0[:, :_V - 1]], axis=1)
        ys2 = jnp.concatenate([ycol2, ycol1, ys0[:, :_V - 2]], axis=1)
        ys_ref[...] = ys2
        dp_a1 = _sum4(jnp.abs(xs1 - ys0)).astype(jnp.float32) * (1.0 / _GAMMA)
        r_a1 = dp_a1 + _softmin3(
            d2, d1, jnp.concatenate([d1[:, 1:], big_col], axis=1))
        da1 = jnp.where(band_mask(p1, bs1), r_a1, _BIG)
        dp_b1 = _sum4(jnp.abs(xs1 - ys1)).astype(jnp.float32) * (1.0 / _GAMMA)
        r_b1 = dp_b1 + _softmin3(d1, shr(da1, big_col), da1)
        db1 = jnp.where(band_mask(p1 + 1, bs1), r_b1, _BIG)
        dp_a2 = _sum4(jnp.abs(xs2 - ys1)).astype(jnp.float32) * (1.0 / _GAMMA)
        r_a2 = dp_a2 + _softmin3(
            da1, db1, jnp.concatenate([db1[:, 1:], big_col], axis=1))
        da2 = jnp.where(band_mask(p1 + 2, bs1 + 1), r_a2, _BIG)
        dp_b2 = _sum4(jnp.abs(xs2 - ys2)).astype(jnp.float32) * (1.0 / _GAMMA)
        r_b2 = dp_b2 + _softmin3(db1, shr(da2, big_col), da2)
        db2 = jnp.where(band_mask(p1 + 3, bs1 + 1), r_b2, _BIG)
        d2_ref[...] = da2
        d1_ref[...] = db2
        return 0

    def phase3_body(t, _):
        hold2_step(663 + 2 * t, _N - _V, 0, _V)
        return 0

    def phase3n_body(t, _):
        hold2_step(663 + 2 * t, _N - _V, 128, _V)
        return 0

    jax.lax.fori_loop(0, 52, phase1n_body, 0)
    jax.lax.fori_loop(52, 75, phase1_body, 0)
    jax.lax.fori_loop(0, 128, pair2_body, 0)
    jax.lax.fori_loop(0, 128, phase3_body, 0)
    jax.lax.fori_loop(128, 180, phase3n_body, 0)
    o_ref[...] = d1_ref[:, _V - 128:]


def _loss_kernel(v_ref, mu_ref, lv_ref, ptc_ref, gtt_ref, att_ref, o_ref):
    v = v_ref[...] * _GAMMA
    vnorm = v[0:32] - 0.5 * (v[32:64] + v[64:96])
    recon = jnp.sum(vnorm) * (1.0 / (32.0 * 32.0))

    mu = mu_ref[...]
    lv = lv_ref[...]
    kl_div = -0.5 * jnp.sum(1.0 + lv - mu * mu - jnp.exp(lv), axis=1)
    kl = jnp.mean(jnp.maximum(kl_div - _KL_FREE_BITS, 0.0))

    gtt = gtt_ref[...]
    gt_trans = jnp.sum(jnp.abs(gtt[:, 1:] - gtt[:, :-1]), axis=1)
    ptc = ptc_ref[...][:, 0]
    aux = jnp.mean((ptc - gt_trans) ** 2)

    att = jax.nn.sigmoid((att_ref[...] - 0.5) * 10.0)
    pred_soft = jnp.sum(jnp.abs(att[:, 1:] - att[:, :-1]), axis=1)
    trans = jnp.mean((pred_soft - gt_trans) ** 2)

    total = _W_RECON * recon + _W_KL * kl + _W_AUX * aux + _W_TRANS * trans
    lane = jax.lax.broadcasted_iota(jnp.int32, (1, 8), 1)
    out = (jnp.where(lane == 0, total, 0.0)
           + jnp.where(lane == 1, recon, 0.0)
           + jnp.where(lane == 2, kl, 0.0)
           + jnp.where(lane == 3, aux, 0.0)
           + jnp.where(lane == 4, trans, 0.0))
    o_ref[...] = out


def kernel(action_trajectory, style_mu, style_logvar,
           predicted_transition_count, ground_truth, interpret=False):
    at = action_trajectory
    gt = ground_truth
    B, N, D = at.shape
    xs = jnp.concatenate([at, at, gt], axis=0)
    ys = jnp.concatenate([gt, at, gt], axis=0)
    xc = xs.transpose(2, 0, 1).reshape(_F * _R, N).astype(jnp.bfloat16)
    yc = ys.transpose(2, 0, 1).reshape(_F * _R, N).astype(jnp.bfloat16)

    last = pl.pallas_call(
        _dp_kernel,
        out_shape=jax.ShapeDtypeStruct((_R, 128), jnp.float32),
        scratch_shapes=[
            pltpu.VMEM((_R, _V), jnp.float32),
            pltpu.VMEM((_R, _V), jnp.float32),
            pltpu.VMEM((4 * _R, _V), jnp.bfloat16),
            pltpu.VMEM((4 * _R, _V), jnp.bfloat16),
        ],
        name="sdtw_dp",
        interpret=interpret,
    )(xc, yc)
    v = last[:, -1].reshape(96, 1)

    out = pl.pallas_call(
        _loss_kernel,
        out_shape=jax.ShapeDtypeStruct((1, 8), jnp.float32),
        name="cvae_losses",
        interpret=interpret,
    )(v, style_mu, style_logvar, predicted_transition_count,
      ground_truth[..., 2], action_trajectory[..., 2])
    return out[0, :5]
